# baseline (speedup 1.0000x reference)
import functools

import jax
import jax.numpy as jnp
from jax import lax
from jax.experimental import pallas as pl
from jax.experimental.pallas import tpu as pltpu
from jax.experimental.pallas import tpu_sc as plsc

_MAX_CTX = 32768
_D = 1024
_NC, _NS = 2, 16
_NW = _NC * _NS
_ROWS_PER_W = _MAX_CTX // _NW
_NBUF = 4
_CHUNK = 16
_NCHUNK = _ROWS_PER_W // _CHUNK

_mesh = plsc.VectorSubcoreMesh(core_axis_name="c", subcore_axis_name="s")


@functools.partial(
    pl.kernel,
    out_type=jax.ShapeDtypeStruct((_MAX_CTX, _D), jnp.float32),
    mesh=_mesh,
    scratch_types=[
        pltpu.VMEM((_NBUF, _CHUNK, _D), jnp.float32),
        pltpu.SemaphoreType.DMA((_NBUF,)),
        pltpu.SemaphoreType.DMA((_NBUF,)),
    ],
)
def _sc_copy(src_hbm, out_hbm, buf, rsem, wsem):
    wid = lax.axis_index("s") * _NC + lax.axis_index("c")
    base = wid * _ROWS_PER_W
    ngroups = _NCHUNK // _NBUF
    for g in range(ngroups):
        for b in range(_NBUF):
            c = base + (g * _NBUF + b) * _CHUNK
            if g > 0:
                pltpu.make_async_copy(
                    buf.at[b], out_hbm.at[pl.ds(c - _NBUF * _CHUNK, _CHUNK)], wsem.at[b]
                ).wait()
            pltpu.make_async_copy(
                src_hbm.at[pl.ds(c, _CHUNK)], buf.at[b], rsem.at[b]
            ).start()
        for b in range(_NBUF):
            c = base + (g * _NBUF + b) * _CHUNK
            pltpu.make_async_copy(
                src_hbm.at[pl.ds(c, _CHUNK)], buf.at[b], rsem.at[b]
            ).wait()
            pltpu.make_async_copy(
                buf.at[b], out_hbm.at[pl.ds(c, _CHUNK)], wsem.at[b]
            ).start()
    for b in range(_NBUF):
        c = base + ((_NCHUNK - _NBUF) + b) * _CHUNK
        pltpu.make_async_copy(
            buf.at[b], out_hbm.at[pl.ds(c, _CHUNK)], wsem.at[b]
        ).wait()


def kernel(h, mem):
    b, l, d = h.shape
    assert b * l == _MAX_CTX and d == _D
    flat = h.reshape(b * l, d)
    new_mem = _sc_copy(flat)
    return (h, new_mem)


# speedup vs baseline: 1.0202x; 1.0202x over previous
import functools

import jax
import jax.numpy as jnp
from jax import lax
from jax.experimental import pallas as pl
from jax.experimental.pallas import tpu as pltpu
from jax.experimental.pallas import tpu_sc as plsc

_MAX_CTX = 32768
_D = 1024
_NC, _NS = 2, 16
_NW = _NC * _NS
_ROWS_PER_W = _MAX_CTX // _NW
_NBUF = 3
_CHUNK = 32
_NCHUNK = _ROWS_PER_W // _CHUNK

_mesh = plsc.VectorSubcoreMesh(core_axis_name="c", subcore_axis_name="s")


@functools.partial(
    pl.kernel,
    out_type=jax.ShapeDtypeStruct((_MAX_CTX, _D), jnp.float32),
    mesh=_mesh,
    scratch_types=[
        pltpu.VMEM((_NBUF, _CHUNK, _D), jnp.float32),
        pltpu.SemaphoreType.DMA((_NBUF,)),
        pltpu.SemaphoreType.DMA((_NBUF,)),
    ],
)
def _sc_copy(src_hbm, out_hbm, buf, rsem, wsem):
    wid = lax.axis_index("s") * _NC + lax.axis_index("c")
    base = wid * _ROWS_PER_W
    ngroups = _NCHUNK // _NBUF
    for g in range(ngroups):
        for b in range(_NBUF):
            c = base + (g * _NBUF + b) * _CHUNK
            if g > 0:
                pltpu.make_async_copy(
                    buf.at[b], out_hbm.at[pl.ds(c - _NBUF * _CHUNK, _CHUNK)], wsem.at[b]
                ).wait()
            pltpu.make_async_copy(
                src_hbm.at[pl.ds(c, _CHUNK)], buf.at[b], rsem.at[b]
            ).start()
        for b in range(_NBUF):
            c = base + (g * _NBUF + b) * _CHUNK
            pltpu.make_async_copy(
                src_hbm.at[pl.ds(c, _CHUNK)], buf.at[b], rsem.at[b]
            ).wait()
            pltpu.make_async_copy(
                buf.at[b], out_hbm.at[pl.ds(c, _CHUNK)], wsem.at[b]
            ).start()
    for b in range(_NBUF):
        c = base + ((_NCHUNK - _NBUF) + b) * _CHUNK
        pltpu.make_async_copy(
            buf.at[b], out_hbm.at[pl.ds(c, _CHUNK)], wsem.at[b]
        ).wait()


def kernel(h, mem):
    b, l, d = h.shape
    assert b * l == _MAX_CTX and d == _D
    flat = h.reshape(b * l, d)
    new_mem = _sc_copy(flat)
    return (h, new_mem)
